# baseline (speedup 1.0000x reference)
import functools

import jax
import jax.numpy as jnp
from jax import lax
from jax.experimental import pallas as pl
from jax.experimental.pallas import tpu as pltpu
from jax.experimental.pallas import tpu_sc as plsc

B, T, D, F = 16, 512, 256, 256
M = 2048
PAD = 8
ZERO_IDX = B * T
NW = 32
B_PER_W = (B * M) // NW
CH = 128
NCHUNK = B_PER_W // CH


def _idx_body(t_ref, gidx_ref):
    b = pl.program_id(0)
    dur = t_ref[0].astype(jnp.float32)
    tt = lax.broadcasted_iota(jnp.int32, (T, T), 0)
    uu = lax.broadcasted_iota(jnp.int32, (T, T), 1)
    tri = (uu <= tt).astype(jnp.float32)
    cum = lax.dot_general(tri, dur, (((1,), (1,)), ((), ())),
                          preferred_element_type=jnp.float32)
    m_row = lax.broadcasted_iota(jnp.int32, (1, M), 1)
    cnt = jnp.sum((cum.astype(jnp.int32) <= m_row).astype(jnp.int32), axis=0,
                  keepdims=True)
    gidx_ref[0] = jnp.where(cnt < T, b * T + cnt, ZERO_IDX)


def _compute_gidx(target):
    t3 = target.reshape(B, 1, T)
    return pl.pallas_call(
        _idx_body,
        grid=(B,),
        in_specs=[pl.BlockSpec((1, 1, T), lambda b: (b, 0, 0))],
        out_specs=pl.BlockSpec((1, 1, M), lambda b: (b, 0, 0)),
        out_shape=jax.ShapeDtypeStruct((B, 1, M), jnp.int32),
    )(t3)


def _gather(table, gidx2):
    mesh = plsc.VectorSubcoreMesh(core_axis_name="c", subcore_axis_name="s")

    NB = 3

    @functools.partial(
        pl.kernel,
        mesh=mesh,
        out_type=jax.ShapeDtypeStruct((B * M, D), jnp.float32),
        scratch_types=[
            pltpu.VMEM((NCHUNK, CH), jnp.int32),
            pltpu.VMEM((NB, CH, D), jnp.float32),
        ] + [pltpu.SemaphoreType.DMA] * (2 * NB),
    )
    def k(table_hbm, idx_hbm, out_hbm, idx_v, bufs, *sems):
        gsems, osems = sems[:NB], sems[NB:]
        wid = lax.axis_index("c") * 16 + lax.axis_index("s")
        pltpu.sync_copy(idx_hbm.at[pl.ds(wid * NCHUNK, NCHUNK)], idx_v)

        def gather(c):
            return pltpu.async_copy(
                table_hbm.at[idx_v.at[c]], bufs.at[c % NB], gsems[c % NB])

        def put(c):
            return pltpu.async_copy(
                bufs.at[c % NB],
                out_hbm.at[pl.ds(wid * B_PER_W + c * CH, CH)], osems[c % NB])

        gcp = [None] * NCHUNK
        ocp = [None] * NCHUNK
        for c in range(NB):
            gcp[c] = gather(c)
        for c in range(NCHUNK):
            gcp[c].wait()
            ocp[c] = put(c)
            if c >= 1 and c + 2 < NCHUNK:
                ocp[c - 1].wait()
                gcp[c + 2] = gather(c + 2)
        ocp[NCHUNK - 2].wait()
        ocp[NCHUNK - 1].wait()

    return k(table, gidx2)


def _dp_body(x_ref, w1_ref, b1_ref, g1_ref, be1_ref, w2_ref, b2_ref, g2_ref,
             be2_ref, lw_ref, lb_ref, dp_ref):
    def conv_ln(h, w_ref, b_ref, g_ref, be_ref):
        row = lax.broadcasted_iota(jnp.int32, (T, 1), 0)
        hm1 = jnp.where(row == 0, 0.0, pltpu.roll(h, 1, 0))
        hp1 = jnp.where(row == T - 1, 0.0, pltpu.roll(h, T - 1, 0))
        hcat = jnp.concatenate([hm1, h, hp1], axis=1)
        y = jnp.dot(hcat, w_ref[...],
                    preferred_element_type=jnp.float32) + b_ref[...]
        y = jnp.maximum(y, 0.0)
        mu = jnp.mean(y, axis=1, keepdims=True)
        var = jnp.mean((y - mu) ** 2, axis=1, keepdims=True)
        return (y - mu) * lax.rsqrt(var + 1e-5) * g_ref[...] + be_ref[...]

    h = conv_ln(x_ref[0], w1_ref, b1_ref, g1_ref, be1_ref)
    h = conv_ln(h, w2_ref, b2_ref, g2_ref, be2_ref)
    dp = lax.dot_general(lw_ref[...], h, (((1,), (1,)), ((), ())),
                         preferred_element_type=jnp.float32)
    dp_ref[0] = dp + lb_ref[0, 0]


def _duration_predictor(x, w1c, b1, g1, be1, w2c, b2, g2, be2, lw, lb):
    full = lambda s: pl.BlockSpec(s, lambda b: tuple(0 for _ in s))
    dp3 = pl.pallas_call(
        _dp_body,
        grid=(B,),
        in_specs=[
            pl.BlockSpec((1, T, D), lambda b: (b, 0, 0)),
            full((3 * D, F)), full((1, F)), full((1, F)), full((1, F)),
            full((3 * F, F)), full((1, F)), full((1, F)), full((1, F)),
            full((1, F)), full((1, 1)),
        ],
        out_specs=pl.BlockSpec((1, 1, T), lambda b: (b, 0, 0)),
        out_shape=jax.ShapeDtypeStruct((B, 1, T), jnp.float32),
    )(x, w1c, b1, g1, be1, w2c, b2, g2, be2, lw, lb)
    return dp3.reshape(B, T)


def kernel(x, target, mel_max_length, conv1_w, conv1_b, ln1_g, ln1_b, conv2_w,
           conv2_b, ln2_g, ln2_b, lin_w, lin_b):
    w1c = conv1_w.transpose(2, 1, 0).reshape(3 * D, F)
    w2c = conv2_w.transpose(2, 1, 0).reshape(3 * F, F)
    b1 = conv1_b.reshape(1, F)
    b2 = conv2_b.reshape(1, F)
    g1, be1 = ln1_g.reshape(1, F), ln1_b.reshape(1, F)
    g2, be2 = ln2_g.reshape(1, F), ln2_b.reshape(1, F)
    lb = lin_b.reshape(1, 1)
    table = jnp.concatenate(
        [x.reshape(B * T, D), jnp.zeros((PAD, D), x.dtype)], axis=0)

    gidx = _compute_gidx(target)
    out_flat = _gather(table, gidx.reshape(NW * NCHUNK, CH))
    out = out_flat.reshape(B, M, D)
    dp = _duration_predictor(x, w1c, b1, g1, be1, w2c, b2, g2, be2, lin_w, lb)
    return (out, dp)


# speedup vs baseline: 1.0032x; 1.0032x over previous
import functools

import jax
import jax.numpy as jnp
from jax import lax
from jax.experimental import pallas as pl
from jax.experimental.pallas import tpu as pltpu
from jax.experimental.pallas import tpu_sc as plsc

B, T, D, F = 16, 512, 256, 256
M = 2048
PAD = 8
ZERO_IDX = B * T
LOC_ZERO = (B // 2) * T
NW = 32
B_PER_W = (B * M) // NW
CH = 64
NCHUNK = B_PER_W // CH


def _idx_body(t_ref, gidx_ref):
    b = pl.program_id(0)
    dur = t_ref[0].astype(jnp.float32)
    tt = lax.broadcasted_iota(jnp.int32, (T, T), 0)
    uu = lax.broadcasted_iota(jnp.int32, (T, T), 1)
    tri = (uu <= tt).astype(jnp.float32)
    cum = lax.dot_general(tri, dur, (((1,), (1,)), ((), ())),
                          preferred_element_type=jnp.float32)
    m_row = lax.broadcasted_iota(jnp.int32, (1, M), 1)
    cnt = jnp.sum((cum.astype(jnp.int32) <= m_row).astype(jnp.int32), axis=0,
                  keepdims=True)
    gidx_ref[0] = jnp.where(cnt < T, b * T + cnt, ZERO_IDX)


def _compute_gidx(target):
    t3 = target.reshape(B, 1, T)
    return pl.pallas_call(
        _idx_body,
        grid=(B,),
        in_specs=[pl.BlockSpec((1, 1, T), lambda b: (b, 0, 0))],
        out_specs=pl.BlockSpec((1, 1, M), lambda b: (b, 0, 0)),
        out_shape=jax.ShapeDtypeStruct((B, 1, M), jnp.int32),
    )(t3)


def _gather(table, gidx2):
    mesh = plsc.VectorSubcoreMesh(core_axis_name="c", subcore_axis_name="s")

    NB = 6

    @functools.partial(
        pl.kernel,
        mesh=mesh,
        out_type=jax.ShapeDtypeStruct((B * M, D), jnp.float32),
        scratch_types=[
            pltpu.VMEM((NCHUNK, CH), jnp.int32),
            pltpu.VMEM((NB, CH, D), jnp.float32),
        ] + [pltpu.SemaphoreType.DMA] * (2 * NB),
    )
    def k(table_hbm, idx_hbm, out_hbm, idx_v, bufs, *sems):
        gsems, osems = sems[:NB], sems[NB:]
        wid = lax.axis_index("c") * 16 + lax.axis_index("s")
        pltpu.sync_copy(idx_hbm.at[pl.ds(wid * NCHUNK, NCHUNK)], idx_v)

        def gather(c):
            return pltpu.async_copy(
                table_hbm.at[idx_v.at[c]], bufs.at[c % NB], gsems[c % NB])

        def put(c):
            return pltpu.async_copy(
                bufs.at[c % NB],
                out_hbm.at[pl.ds(wid * B_PER_W + c * CH, CH)], osems[c % NB])

        gcp = [None] * NCHUNK
        ocp = [None] * NCHUNK
        for c in range(NB):
            gcp[c] = gather(c)
        for c in range(NCHUNK):
            gcp[c].wait()
            ocp[c] = put(c)
            if c >= 1 and c + NB - 1 < NCHUNK:
                ocp[c - 1].wait()
                gcp[c + NB - 1] = gather(c + NB - 1)
        for c in range(max(0, NCHUNK - NB), NCHUNK):
            ocp[c].wait()

    return k(table, gidx2)


def _dp_body(x_ref, w1_ref, b1_ref, g1_ref, be1_ref, w2_ref, b2_ref, g2_ref,
             be2_ref, lw_ref, lb_ref, dp_ref):
    def conv_ln(h, w_ref, b_ref, g_ref, be_ref):
        row = lax.broadcasted_iota(jnp.int32, (T, 1), 0)
        hm1 = jnp.where(row == 0, 0.0, pltpu.roll(h, 1, 0))
        hp1 = jnp.where(row == T - 1, 0.0, pltpu.roll(h, T - 1, 0))
        hcat = jnp.concatenate([hm1, h, hp1], axis=1)
        y = jnp.dot(hcat, w_ref[...],
                    preferred_element_type=jnp.float32) + b_ref[...]
        y = jnp.maximum(y, 0.0)
        mu = jnp.mean(y, axis=1, keepdims=True)
        var = jnp.mean((y - mu) ** 2, axis=1, keepdims=True)
        return (y - mu) * lax.rsqrt(var + 1e-5) * g_ref[...] + be_ref[...]

    h = conv_ln(x_ref[0], w1_ref, b1_ref, g1_ref, be1_ref)
    h = conv_ln(h, w2_ref, b2_ref, g2_ref, be2_ref)
    dp = lax.dot_general(lw_ref[...], h, (((1,), (1,)), ((), ())),
                         preferred_element_type=jnp.float32)
    dp_ref[0] = dp + lb_ref[0, 0]


def _duration_predictor(x, w1c, b1, g1, be1, w2c, b2, g2, be2, lw, lb):
    full = lambda s: pl.BlockSpec(s, lambda b: tuple(0 for _ in s))
    dp3 = pl.pallas_call(
        _dp_body,
        grid=(B,),
        in_specs=[
            pl.BlockSpec((1, T, D), lambda b: (b, 0, 0)),
            full((3 * D, F)), full((1, F)), full((1, F)), full((1, F)),
            full((3 * F, F)), full((1, F)), full((1, F)), full((1, F)),
            full((1, F)), full((1, 1)),
        ],
        out_specs=pl.BlockSpec((1, 1, T), lambda b: (b, 0, 0)),
        out_shape=jax.ShapeDtypeStruct((B, 1, T), jnp.float32),
    )(x, w1c, b1, g1, be1, w2c, b2, g2, be2, lw, lb)
    return dp3.reshape(B, T)


def kernel(x, target, mel_max_length, conv1_w, conv1_b, ln1_g, ln1_b, conv2_w,
           conv2_b, ln2_g, ln2_b, lin_w, lin_b):
    w1c = conv1_w.transpose(2, 1, 0).reshape(3 * D, F)
    w2c = conv2_w.transpose(2, 1, 0).reshape(3 * F, F)
    b1 = conv1_b.reshape(1, F)
    b2 = conv2_b.reshape(1, F)
    g1, be1 = ln1_g.reshape(1, F), ln1_b.reshape(1, F)
    g2, be2 = ln2_g.reshape(1, F), ln2_b.reshape(1, F)
    lb = lin_b.reshape(1, 1)
    table = jnp.concatenate(
        [x.reshape(B * T, D), jnp.zeros((PAD, D), x.dtype)], axis=0)

    gidx = _compute_gidx(target)
    out_flat = _gather(table, gidx.reshape(NW * NCHUNK, CH))
    out = out_flat.reshape(B, M, D)
    dp = _duration_predictor(x, w1c, b1, g1, be1, w2c, b2, g2, be2, lin_w, lb)
    return (out, dp)


# speedup vs baseline: 3.0555x; 3.0456x over previous
import functools

import jax
import jax.numpy as jnp
from jax import lax
from jax.experimental import pallas as pl
from jax.experimental.pallas import tpu as pltpu
from jax.experimental.pallas import tpu_sc as plsc

B, T, D, F = 16, 512, 256, 256
M = 2048
PAD = 2048
ZERO_IDX = B * T
NW = 32
B_PER_W = (B * M) // NW
CH = 64
NCHUNK = B_PER_W // CH


def _idx_body(t_ref, gidx_ref):
    b = pl.program_id(0)
    dur = t_ref[0].astype(jnp.float32)
    tt = lax.broadcasted_iota(jnp.int32, (T, T), 0)
    uu = lax.broadcasted_iota(jnp.int32, (T, T), 1)
    tri = (uu <= tt).astype(jnp.float32)
    cum = lax.dot_general(tri, dur, (((1,), (1,)), ((), ())),
                          preferred_element_type=jnp.float32)
    m_row = lax.broadcasted_iota(jnp.int32, (1, M), 1)
    cnt = jnp.sum((cum.astype(jnp.int32) <= m_row).astype(jnp.int32), axis=0,
                  keepdims=True)
    gidx_ref[0] = jnp.where(cnt < T, b * T + cnt, ZERO_IDX + m_row)


def _compute_gidx(target):
    t3 = target.reshape(B, 1, T)
    return pl.pallas_call(
        _idx_body,
        grid=(B,),
        in_specs=[pl.BlockSpec((1, 1, T), lambda b: (b, 0, 0))],
        out_specs=pl.BlockSpec((1, 1, M), lambda b: (b, 0, 0)),
        out_shape=jax.ShapeDtypeStruct((B, 1, M), jnp.int32),
    )(t3)


def _gather(table, gidx2):
    mesh = plsc.VectorSubcoreMesh(core_axis_name="c", subcore_axis_name="s")

    NB = 6

    @functools.partial(
        pl.kernel,
        mesh=mesh,
        out_type=jax.ShapeDtypeStruct((B * M, D), jnp.float32),
        scratch_types=[pltpu.VMEM((CH,), jnp.int32)] * NCHUNK + [
            pltpu.VMEM((NB, CH, D), jnp.float32),
        ] + [pltpu.SemaphoreType.DMA] * (2 * NB),
    )
    def k(table_hbm, idx_hbm, out_hbm, *refs):
        idx_refs = refs[:NCHUNK]
        bufs = refs[NCHUNK]
        sems = refs[NCHUNK + 1:]
        gsems, osems = sems[:NB], sems[NB:]
        wid = lax.axis_index("c") * 16 + lax.axis_index("s")
        for c in range(NCHUNK):
            pltpu.sync_copy(idx_hbm.at[wid * NCHUNK + c], idx_refs[c])

        def gather(c):
            return pltpu.async_copy(
                table_hbm.at[idx_refs[c]], bufs.at[c % NB], gsems[c % NB])

        def put(c):
            return pltpu.async_copy(
                bufs.at[c % NB],
                out_hbm.at[pl.ds(wid * B_PER_W + c * CH, CH)], osems[c % NB])

        gcp = [None] * NCHUNK
        ocp = [None] * NCHUNK
        for c in range(NB):
            gcp[c] = gather(c)
        for c in range(NCHUNK):
            gcp[c].wait()
            ocp[c] = put(c)
            if c >= 1 and c + NB - 1 < NCHUNK:
                ocp[c - 1].wait()
                gcp[c + NB - 1] = gather(c + NB - 1)
        for c in range(max(0, NCHUNK - NB), NCHUNK):
            ocp[c].wait()

    return k(table, gidx2)


def _dp_body(x_ref, w1_ref, b1_ref, g1_ref, be1_ref, w2_ref, b2_ref, g2_ref,
             be2_ref, lw_ref, lb_ref, dp_ref):
    def conv_ln(h, w_ref, b_ref, g_ref, be_ref):
        row = lax.broadcasted_iota(jnp.int32, (T, 1), 0)
        hm1 = jnp.where(row == 0, 0.0, pltpu.roll(h, 1, 0))
        hp1 = jnp.where(row == T - 1, 0.0, pltpu.roll(h, T - 1, 0))
        hcat = jnp.concatenate([hm1, h, hp1], axis=1)
        y = jnp.dot(hcat, w_ref[...],
                    preferred_element_type=jnp.float32) + b_ref[...]
        y = jnp.maximum(y, 0.0)
        mu = jnp.mean(y, axis=1, keepdims=True)
        var = jnp.mean((y - mu) ** 2, axis=1, keepdims=True)
        return (y - mu) * lax.rsqrt(var + 1e-5) * g_ref[...] + be_ref[...]

    h = conv_ln(x_ref[0], w1_ref, b1_ref, g1_ref, be1_ref)
    h = conv_ln(h, w2_ref, b2_ref, g2_ref, be2_ref)
    dp = lax.dot_general(lw_ref[...], h, (((1,), (1,)), ((), ())),
                         preferred_element_type=jnp.float32)
    dp_ref[0] = dp + lb_ref[0, 0]


def _duration_predictor(x, w1c, b1, g1, be1, w2c, b2, g2, be2, lw, lb):
    full = lambda s: pl.BlockSpec(s, lambda b: tuple(0 for _ in s))
    dp3 = pl.pallas_call(
        _dp_body,
        grid=(B,),
        in_specs=[
            pl.BlockSpec((1, T, D), lambda b: (b, 0, 0)),
            full((3 * D, F)), full((1, F)), full((1, F)), full((1, F)),
            full((3 * F, F)), full((1, F)), full((1, F)), full((1, F)),
            full((1, F)), full((1, 1)),
        ],
        out_specs=pl.BlockSpec((1, 1, T), lambda b: (b, 0, 0)),
        out_shape=jax.ShapeDtypeStruct((B, 1, T), jnp.float32),
    )(x, w1c, b1, g1, be1, w2c, b2, g2, be2, lw, lb)
    return dp3.reshape(B, T)


def kernel(x, target, mel_max_length, conv1_w, conv1_b, ln1_g, ln1_b, conv2_w,
           conv2_b, ln2_g, ln2_b, lin_w, lin_b):
    w1c = conv1_w.transpose(2, 1, 0).reshape(3 * D, F)
    w2c = conv2_w.transpose(2, 1, 0).reshape(3 * F, F)
    b1 = conv1_b.reshape(1, F)
    b2 = conv2_b.reshape(1, F)
    g1, be1 = ln1_g.reshape(1, F), ln1_b.reshape(1, F)
    g2, be2 = ln2_g.reshape(1, F), ln2_b.reshape(1, F)
    lb = lin_b.reshape(1, 1)
    table = jnp.concatenate(
        [x.reshape(B * T, D), jnp.zeros((PAD, D), x.dtype)], axis=0)

    gidx = _compute_gidx(target)
    out_flat = _gather(table, gidx.reshape(NW * NCHUNK, CH))
    out = out_flat.reshape(B, M, D)
    dp = _duration_predictor(x, w1c, b1, g1, be1, w2c, b2, g2, be2, lin_w, lb)
    return (out, dp)


# speedup vs baseline: 3.3523x; 1.0972x over previous
import functools

import jax
import jax.numpy as jnp
from jax import lax
from jax.experimental import pallas as pl
from jax.experimental.pallas import tpu as pltpu
from jax.experimental.pallas import tpu_sc as plsc

B, T, D, F = 16, 512, 256, 256
M = 2048
NW = 32
B_PER_W = (B * M) // NW
CH = 64
NCHUNK = B_PER_W // CH
NB = 6
L = 16


def _idx_body(t_ref, gidx_ref, tot_ref):
    b = pl.program_id(0)
    dur = t_ref[0].astype(jnp.float32)
    tt = lax.broadcasted_iota(jnp.int32, (T, T), 0)
    uu = lax.broadcasted_iota(jnp.int32, (T, T), 1)
    tri = (uu <= tt).astype(jnp.float32)
    cum = lax.dot_general(tri, dur, (((1,), (1,)), ((), ())),
                          preferred_element_type=jnp.float32)
    cum_i = cum.astype(jnp.int32)
    m_row = lax.broadcasted_iota(jnp.int32, (1, M), 1)
    cnt = jnp.sum((cum_i <= m_row).astype(jnp.int32), axis=0,
                  keepdims=True)
    gidx_ref[0] = jnp.where(cnt < T, b * T + cnt,
                            b * T + (m_row & (T - 1)))
    tot_ref[0] = jnp.broadcast_to(jnp.max(cum_i), (1, 128))


def _compute_gidx(target):
    t3 = target.reshape(B, 1, T)
    return pl.pallas_call(
        _idx_body,
        grid=(B,),
        in_specs=[pl.BlockSpec((1, 1, T), lambda b: (b, 0, 0))],
        out_specs=[pl.BlockSpec((1, 1, M), lambda b: (b, 0, 0)),
                   pl.BlockSpec((1, 1, 128), lambda b: (b, 0, 0))],
        out_shape=[jax.ShapeDtypeStruct((B, 1, M), jnp.int32),
                   jax.ShapeDtypeStruct((B, 1, 128), jnp.int32)],
    )(t3)


def _gather(table, gidx2, tot2):
    mesh = plsc.VectorSubcoreMesh(core_axis_name="c", subcore_axis_name="s")

    @functools.partial(
        pl.kernel,
        mesh=mesh,
        out_type=jax.ShapeDtypeStruct((B * M, D), jnp.float32),
        scratch_types=[pltpu.VMEM((CH,), jnp.int32)] * NCHUNK + [
            pltpu.VMEM((128,), jnp.int32),
            pltpu.VMEM((NB, CH, D), jnp.float32),
        ] + [pltpu.SemaphoreType.DMA] * (2 * NB),
    )
    def k(table_hbm, idx_hbm, tot_hbm, out_hbm, *refs):
        idx_refs = refs[:NCHUNK]
        tot_v = refs[NCHUNK]
        bufs = refs[NCHUNK + 1]
        sems = refs[NCHUNK + 2:]
        gsems, osems = sems[:NB], sems[NB:]
        wid = lax.axis_index("c") * 16 + lax.axis_index("s")
        b = wid // 2
        m0 = (wid % 2) * B_PER_W
        pltpu.sync_copy(tot_hbm.at[b], tot_v)
        for c in range(NCHUNK):
            pltpu.sync_copy(idx_hbm.at[wid * NCHUNK + c], idx_refs[c])
        total = tot_v[pl.ds(0, L)][0]
        vlim = jnp.clip(total - m0, 0, B_PER_W)

        def gather(c):
            return pltpu.async_copy(
                table_hbm.at[idx_refs[c]], bufs.at[c % NB], gsems[c % NB])

        def put(c):
            return pltpu.async_copy(
                bufs.at[c % NB],
                out_hbm.at[pl.ds(wid * B_PER_W + c * CH, CH)], osems[c % NB])

        zrow = jnp.zeros((L,), jnp.float32)

        def zero_tail(c):
            lo = jnp.clip(vlim - c * CH, 0, CH)

            def body(i, _):
                for gg in range(D // L):
                    bufs[c % NB, i, pl.ds(gg * L, L)] = zrow
                return 0

            lax.fori_loop(lo, CH, body, 0)

        gcp = [None] * NCHUNK
        ocp = [None] * NCHUNK
        for c in range(NB):
            gcp[c] = gather(c)
        for c in range(NCHUNK):
            gcp[c].wait()
            zero_tail(c)
            ocp[c] = put(c)
            if c >= 1 and c + NB - 1 < NCHUNK:
                ocp[c - 1].wait()
                gcp[c + NB - 1] = gather(c + NB - 1)
        for c in range(max(0, NCHUNK - NB), NCHUNK):
            ocp[c].wait()

    return k(table, gidx2, tot2)


def _dp_body(x_ref, w1_ref, b1_ref, g1_ref, be1_ref, w2_ref, b2_ref, g2_ref,
             be2_ref, lw_ref, lb_ref, dp_ref):
    def conv_ln(h, w_ref, b_ref, g_ref, be_ref):
        row = lax.broadcasted_iota(jnp.int32, (T, 1), 0)
        hm1 = jnp.where(row == 0, 0.0, pltpu.roll(h, 1, 0))
        hp1 = jnp.where(row == T - 1, 0.0, pltpu.roll(h, T - 1, 0))
        hcat = jnp.concatenate([hm1, h, hp1], axis=1)
        y = jnp.dot(hcat, w_ref[...],
                    preferred_element_type=jnp.float32) + b_ref[...]
        y = jnp.maximum(y, 0.0)
        mu = jnp.mean(y, axis=1, keepdims=True)
        var = jnp.mean((y - mu) ** 2, axis=1, keepdims=True)
        return (y - mu) * lax.rsqrt(var + 1e-5) * g_ref[...] + be_ref[...]

    h = conv_ln(x_ref[0], w1_ref, b1_ref, g1_ref, be1_ref)
    h = conv_ln(h, w2_ref, b2_ref, g2_ref, be2_ref)
    dp = lax.dot_general(lw_ref[...], h, (((1,), (1,)), ((), ())),
                         preferred_element_type=jnp.float32)
    dp_ref[0] = dp + lb_ref[0, 0]


def _duration_predictor(x, w1c, b1, g1, be1, w2c, b2, g2, be2, lw, lb):
    full = lambda s: pl.BlockSpec(s, lambda b: tuple(0 for _ in s))
    dp3 = pl.pallas_call(
        _dp_body,
        grid=(B,),
        in_specs=[
            pl.BlockSpec((1, T, D), lambda b: (b, 0, 0)),
            full((3 * D, F)), full((1, F)), full((1, F)), full((1, F)),
            full((3 * F, F)), full((1, F)), full((1, F)), full((1, F)),
            full((1, F)), full((1, 1)),
        ],
        out_specs=pl.BlockSpec((1, 1, T), lambda b: (b, 0, 0)),
        out_shape=jax.ShapeDtypeStruct((B, 1, T), jnp.float32),
    )(x, w1c, b1, g1, be1, w2c, b2, g2, be2, lw, lb)
    return dp3.reshape(B, T)


def kernel(x, target, mel_max_length, conv1_w, conv1_b, ln1_g, ln1_b, conv2_w,
           conv2_b, ln2_g, ln2_b, lin_w, lin_b):
    w1c = conv1_w.transpose(2, 1, 0).reshape(3 * D, F)
    w2c = conv2_w.transpose(2, 1, 0).reshape(3 * F, F)
    b1 = conv1_b.reshape(1, F)
    b2 = conv2_b.reshape(1, F)
    g1, be1 = ln1_g.reshape(1, F), ln1_b.reshape(1, F)
    g2, be2 = ln2_g.reshape(1, F), ln2_b.reshape(1, F)
    lb = lin_b.reshape(1, 1)

    gidx, tot = _compute_gidx(target)
    out_flat = _gather(x.reshape(B * T, D), gidx.reshape(NW * NCHUNK, CH),
                       tot.reshape(B, 128))
    out = out_flat.reshape(B, M, D)
    dp = _duration_predictor(x, w1c, b1, g1, be1, w2c, b2, g2, be2, lin_w, lb)
    return (out, dp)


# speedup vs baseline: 3.4096x; 1.0171x over previous
import functools

import jax
import jax.numpy as jnp
from jax import lax
from jax.experimental import pallas as pl
from jax.experimental.pallas import tpu as pltpu
from jax.experimental.pallas import tpu_sc as plsc

B, T, D, F = 16, 512, 256, 256
M = 2048
NW = 32
B_PER_W = (B * M) // NW
CH = 64
NCHUNK = B_PER_W // CH
NB = 6
L = 16


def _idx_body(t_ref, gidx_ref, tot_ref):
    b = pl.program_id(0)
    dur = t_ref[0].astype(jnp.float32)
    tt = lax.broadcasted_iota(jnp.int32, (T, T), 0)
    uu = lax.broadcasted_iota(jnp.int32, (T, T), 1)
    tri = (uu <= tt).astype(jnp.float32)
    cum = lax.dot_general(tri, dur, (((1,), (1,)), ((), ())),
                          preferred_element_type=jnp.float32)
    cum_i = cum.astype(jnp.int32)
    m_row = lax.broadcasted_iota(jnp.int32, (1, M), 1)
    cmp_f = jnp.where(cum <= m_row.astype(jnp.float32), 1.0, 0.0)
    ones_row = jnp.full((1, T), 1.0, jnp.float32)
    cnt = jnp.dot(ones_row, cmp_f,
                  preferred_element_type=jnp.float32).astype(jnp.int32)
    gidx_ref[0] = jnp.where(cnt < T, b * T + cnt,
                            b * T + (m_row & (T - 1)))
    tot_ref[0] = jnp.broadcast_to(jnp.max(cum_i), (1, 128))


def _compute_gidx(target):
    t3 = target.reshape(B, 1, T)
    return pl.pallas_call(
        _idx_body,
        grid=(B,),
        in_specs=[pl.BlockSpec((1, 1, T), lambda b: (b, 0, 0))],
        out_specs=[pl.BlockSpec((1, 1, M), lambda b: (b, 0, 0)),
                   pl.BlockSpec((1, 1, 128), lambda b: (b, 0, 0))],
        out_shape=[jax.ShapeDtypeStruct((B, 1, M), jnp.int32),
                   jax.ShapeDtypeStruct((B, 1, 128), jnp.int32)],
    )(t3)


def _gather(table, gidx2, tot2):
    mesh = plsc.VectorSubcoreMesh(core_axis_name="c", subcore_axis_name="s")

    @functools.partial(
        pl.kernel,
        mesh=mesh,
        out_type=jax.ShapeDtypeStruct((B * M, D), jnp.float32),
        scratch_types=[pltpu.VMEM((CH,), jnp.int32)] * NCHUNK + [
            pltpu.VMEM((128,), jnp.int32),
            pltpu.VMEM((NB, CH, D), jnp.float32),
        ] + [pltpu.SemaphoreType.DMA] * (2 * NB),
    )
    def k(table_hbm, idx_hbm, tot_hbm, out_hbm, *refs):
        idx_refs = refs[:NCHUNK]
        tot_v = refs[NCHUNK]
        bufs = refs[NCHUNK + 1]
        sems = refs[NCHUNK + 2:]
        gsems, osems = sems[:NB], sems[NB:]
        wid = lax.axis_index("c") * 16 + lax.axis_index("s")
        b = wid // 2
        m0 = (wid % 2) * B_PER_W
        pltpu.sync_copy(tot_hbm.at[b], tot_v)
        for c in range(NCHUNK):
            pltpu.sync_copy(idx_hbm.at[wid * NCHUNK + c], idx_refs[c])
        total = tot_v[pl.ds(0, L)][0]
        vlim = jnp.clip(total - m0, 0, B_PER_W)

        def gather(c):
            return pltpu.async_copy(
                table_hbm.at[idx_refs[c]], bufs.at[c % NB], gsems[c % NB])

        def put(c):
            return pltpu.async_copy(
                bufs.at[c % NB],
                out_hbm.at[pl.ds(wid * B_PER_W + c * CH, CH)], osems[c % NB])

        zrow = jnp.zeros((L,), jnp.float32)

        def zero_tail(c):
            lo = jnp.clip(vlim - c * CH, 0, CH)

            def body(i, _):
                for gg in range(D // L):
                    bufs[c % NB, i, pl.ds(gg * L, L)] = zrow
                return 0

            lax.fori_loop(lo, CH, body, 0)

        gcp = [None] * NCHUNK
        ocp = [None] * NCHUNK
        for c in range(NB):
            gcp[c] = gather(c)
        for c in range(NCHUNK):
            gcp[c].wait()
            zero_tail(c)
            ocp[c] = put(c)
            if c >= 1 and c + NB - 1 < NCHUNK:
                ocp[c - 1].wait()
                gcp[c + NB - 1] = gather(c + NB - 1)
        for c in range(max(0, NCHUNK - NB), NCHUNK):
            ocp[c].wait()

    return k(table, gidx2, tot2)


def _dp_body(x_ref, w1_ref, b1_ref, g1_ref, be1_ref, w2_ref, b2_ref, g2_ref,
             be2_ref, lw_ref, lb_ref, dp_ref):
    def conv_ln(h, w_ref, b_ref, g_ref, be_ref):
        row = lax.broadcasted_iota(jnp.int32, (T, 1), 0)
        hm1 = jnp.where(row == 0, 0.0, pltpu.roll(h, 1, 0))
        hp1 = jnp.where(row == T - 1, 0.0, pltpu.roll(h, T - 1, 0))
        hcat = jnp.concatenate([hm1, h, hp1], axis=1)
        y = jnp.dot(hcat, w_ref[...],
                    preferred_element_type=jnp.float32) + b_ref[...]
        y = jnp.maximum(y, 0.0)
        mu = jnp.mean(y, axis=1, keepdims=True)
        var = jnp.mean((y - mu) ** 2, axis=1, keepdims=True)
        return (y - mu) * lax.rsqrt(var + 1e-5) * g_ref[...] + be_ref[...]

    h = conv_ln(x_ref[0], w1_ref, b1_ref, g1_ref, be1_ref)
    h = conv_ln(h, w2_ref, b2_ref, g2_ref, be2_ref)
    dp = lax.dot_general(lw_ref[...], h, (((1,), (1,)), ((), ())),
                         preferred_element_type=jnp.float32)
    dp_ref[0] = dp + lb_ref[0, 0]


def _duration_predictor(x, w1c, b1, g1, be1, w2c, b2, g2, be2, lw, lb):
    full = lambda s: pl.BlockSpec(s, lambda b: tuple(0 for _ in s))
    dp3 = pl.pallas_call(
        _dp_body,
        grid=(B,),
        in_specs=[
            pl.BlockSpec((1, T, D), lambda b: (b, 0, 0)),
            full((3 * D, F)), full((1, F)), full((1, F)), full((1, F)),
            full((3 * F, F)), full((1, F)), full((1, F)), full((1, F)),
            full((1, F)), full((1, 1)),
        ],
        out_specs=pl.BlockSpec((1, 1, T), lambda b: (b, 0, 0)),
        out_shape=jax.ShapeDtypeStruct((B, 1, T), jnp.float32),
    )(x, w1c, b1, g1, be1, w2c, b2, g2, be2, lw, lb)
    return dp3.reshape(B, T)


def kernel(x, target, mel_max_length, conv1_w, conv1_b, ln1_g, ln1_b, conv2_w,
           conv2_b, ln2_g, ln2_b, lin_w, lin_b):
    w1c = conv1_w.transpose(2, 1, 0).reshape(3 * D, F)
    w2c = conv2_w.transpose(2, 1, 0).reshape(3 * F, F)
    b1 = conv1_b.reshape(1, F)
    b2 = conv2_b.reshape(1, F)
    g1, be1 = ln1_g.reshape(1, F), ln1_b.reshape(1, F)
    g2, be2 = ln2_g.reshape(1, F), ln2_b.reshape(1, F)
    lb = lin_b.reshape(1, 1)

    gidx, tot = _compute_gidx(target)
    out_flat = _gather(x.reshape(B * T, D), gidx.reshape(NW * NCHUNK, CH),
                       tot.reshape(B, 128))
    out = out_flat.reshape(B, M, D)
    dp = _duration_predictor(x, w1c, b1, g1, be1, w2c, b2, g2, be2, lin_w, lb)
    return (out, dp)
